# Initial kernel scaffold; baseline (speedup 1.0000x reference)
import jax
import jax.numpy as jnp
from jax.experimental import pallas as pl


def kernel(x, edge_index, edge_type, W_sl, b_sl, W1, b1, gamma, beta, W2, b2):
    raise NotImplementedError("write your pallas kernel here")


# speedup vs baseline: 3.1961x; 3.1961x over previous
import functools

import jax
import jax.numpy as jnp
from jax import lax
from jax.experimental import pallas as pl
from jax.experimental.pallas import tpu as pltpu
from jax.experimental.pallas import tpu_sc as plsc

_N = 10000
_E = 320000
_D = 128
_R = 4
_BN_EPS = 1e-5

_NC = 2
_NS = 16
_CH = 80
_EPT = _E // _NS
_NCHUNK = _EPT // _CH
_ACC_ROWS = 10240
_ZROWS = _ACC_ROWS // _NS


def _sc_body(x_hbm, src_hbm, cidx_hbm, zeros_hbm, agg_hbm,
             acc, idx_buf, cidx_buf, scat_buf, rows_buf, sem):
    c = lax.axis_index("c")
    s = lax.axis_index("s")
    base_e = s * _EPT

    for p in range(2):
        r = c * 2 + p
        pltpu.sync_copy(zeros_hbm, acc.at[pl.ds(s * _ZROWS, _ZROWS)])
        plsc.subcore_barrier()

        def chunk_body(k, carry):
            off = pl.multiple_of(base_e + k * _CH, 8)
            pltpu.sync_copy(src_hbm.at[pl.ds(off, _CH)], idx_buf)
            pltpu.sync_copy(cidx_hbm.at[pl.ds(off, _CH)], cidx_buf)
            for j in range(_CH // 16):
                v = cidx_buf[pl.ds(j * 16, 16)]
                local = v - r * _N
                ok = (local >= 0) & (local < _N)
                scat_buf[pl.ds(j * 16, 16)] = jnp.where(ok, local, _N)
            pltpu.async_copy(x_hbm.at[idx_buf], rows_buf, sem).wait()
            pltpu.sync_copy(rows_buf, acc.at[scat_buf], add=True)
            return carry

        lax.fori_loop(0, _NCHUNK, chunk_body, 0)
        plsc.subcore_barrier()

        pltpu.sync_copy(acc.at[pl.ds(s * _ZROWS, _ZROWS)],
                        agg_hbm.at[pl.ds(r * _ACC_ROWS + s * _ZROWS, _ZROWS)])
        plsc.subcore_barrier()


def _sc_aggregate(x, src, cidx, zeros_blk):
    mesh = plsc.VectorSubcoreMesh(core_axis_name="c", subcore_axis_name="s")
    kern = functools.partial(
        pl.kernel,
        mesh=mesh,
        out_type=jax.ShapeDtypeStruct((_R * _ACC_ROWS, _D), jnp.float32),
        scratch_types=[
            pltpu.VMEM_SHARED((_ACC_ROWS, _D), jnp.float32),
            pltpu.VMEM((_CH,), jnp.int32),
            pltpu.VMEM((_CH,), jnp.int32),
            pltpu.VMEM((_CH,), jnp.int32),
            pltpu.VMEM((_CH, _D), jnp.float32),
            pltpu.SemaphoreType.DMA,
        ],
    )(_sc_body)
    return kern(x, src, cidx, zeros_blk)


def _dot(a, b):
    return lax.dot_general(a, b, (((1,), (0,)), ((), ())),
                           precision=lax.Precision.HIGHEST,
                           preferred_element_type=jnp.float32)


def _tc_body(x_ref, agg_ref, wsl_ref, bsl_ref, w1_ref, b1_ref, g_ref, be_ref,
             w2_ref, b2_ref, out_ref):
    r = pl.program_id(0)
    x = x_ref[...]
    h = x + agg_ref[0]
    h = _dot(h, w1_ref[0]) + b1_ref[0]
    mean = jnp.mean(h, axis=0, keepdims=True)
    d = h - mean
    var = jnp.mean(d * d, axis=0, keepdims=True)
    hn = d * lax.rsqrt(var + _BN_EPS) * g_ref[0] + be_ref[0]
    hn = jnp.maximum(hn, 0.0)
    h2 = _dot(hn, w2_ref[0]) + b2_ref[0]

    @pl.when(r == 0)
    def _():
        out_ref[...] = _dot(x, wsl_ref[...]) + bsl_ref[...] + h2

    @pl.when(r != 0)
    def _():
        out_ref[...] = out_ref[...] + h2


def _tc_mlp(x, agg4, W_sl, b_sl, W1, b1, gamma, beta, W2, b2):
    full2 = pl.BlockSpec((_N, _D), lambda r: (0, 0))
    per_rel_vec = pl.BlockSpec((1, 1, _D), lambda r: (r, 0, 0))
    return pl.pallas_call(
        _tc_body,
        grid=(_R,),
        in_specs=[
            full2,
            pl.BlockSpec((1, _N, _D), lambda r: (r, 0, 0)),
            pl.BlockSpec((_D, _D), lambda r: (0, 0)),
            pl.BlockSpec((1, _D), lambda r: (0, 0)),
            pl.BlockSpec((1, _D, _D), lambda r: (r, 0, 0)),
            per_rel_vec,
            per_rel_vec,
            per_rel_vec,
            pl.BlockSpec((1, _D, _D), lambda r: (r, 0, 0)),
            per_rel_vec,
        ],
        out_specs=full2,
        out_shape=jax.ShapeDtypeStruct((_N, _D), jnp.float32),
        compiler_params=pltpu.CompilerParams(vmem_limit_bytes=100 * 1024 * 1024),
    )(x, agg4, W_sl, b_sl.reshape(1, _D),
      W1, b1.reshape(_R, 1, _D), gamma.reshape(_R, 1, _D),
      beta.reshape(_R, 1, _D), W2, b2.reshape(_R, 1, _D))


def kernel(x, edge_index, edge_type, W_sl, b_sl, W1, b1, gamma, beta, W2, b2):
    src = edge_index[0]
    cidx = edge_type * jnp.int32(_N) + edge_index[1]
    zeros_blk = jnp.zeros((_ZROWS, _D), jnp.float32)
    agg = _sc_aggregate(x, src, cidx, zeros_blk)
    agg4 = agg.reshape(_R, _ACC_ROWS, _D)
    return _tc_mlp(x, agg4, W_sl, b_sl, W1, b1, gamma, beta, W2, b2)


# speedup vs baseline: 5.5034x; 1.7219x over previous
import functools

import jax
import jax.numpy as jnp
from jax import lax
from jax.experimental import pallas as pl
from jax.experimental.pallas import tpu as pltpu
from jax.experimental.pallas import tpu_sc as plsc

_N = 10000
_E = 320000
_D = 128
_R = 4
_BN_EPS = 1e-5

_NC = 2
_NS = 16
_CH = 80
_EPT = _E // _NS
_NCHUNK = _EPT // _CH
_ACC_ROWS = 10240
_ZROWS = _ACC_ROWS // _NS


_BLKC = 10
_IB = _BLKC * _CH
_NBLK = _NCHUNK // _BLKC


def _sc_body(x_hbm, src_hbm, cidx_hbm, zeros_hbm, agg_hbm,
             acc, src_blk, cidx_blk, scat0, scat1, rows0, rows1, gsem, ssem):
    c = lax.axis_index("c")
    s = lax.axis_index("s")
    base_e = s * _EPT
    rows = (rows0, rows1)
    scat = (scat0, scat1)

    def load_blk(off):
        off = pl.multiple_of(off, 8)
        pltpu.sync_copy(src_hbm.at[pl.ds(off, _IB)], src_blk)
        pltpu.sync_copy(cidx_hbm.at[pl.ds(off, _IB)], cidx_blk)

    for p in range(2):
        r = c * 2 + p
        rbase = r * _N

        def compute_scat(jc, sbuf):
            for j in range(_CH // 16):
                v = cidx_blk[pl.ds(jc * _CH + j * 16, 16)]
                local = v - rbase
                ok = (local >= 0) & (local < _N)
                sbuf[pl.ds(j * 16, 16)] = jnp.where(ok, local, _N)

        def start_gather(jc, rbuf):
            pltpu.async_copy(x_hbm.at[src_blk.at[pl.ds(jc * _CH, _CH)]],
                             rbuf, gsem)

        def wait_gather(rbuf):
            pltpu.make_async_copy(x_hbm.at[src_blk.at[pl.ds(0, _CH)]],
                                  rbuf, gsem).wait()

        def wait_scatter(i):
            pltpu.make_async_copy(rows[i], acc.at[scat[i]], ssem).wait()

        pltpu.sync_copy(zeros_hbm, acc.at[pl.ds(s * _ZROWS, _ZROWS)])
        plsc.subcore_barrier()

        load_blk(base_e)
        compute_scat(0, scat[0])
        start_gather(0, rows[0])

        def block_body(b, carry):
            for j in range(_BLKC):
                cur = j % 2
                nxt = 1 - cur
                wait_gather(rows[cur])
                pltpu.async_copy(rows[cur], acc.at[scat[cur]], ssem, add=True)
                if j == 0:
                    @pl.when(b > 0)
                    def _():
                        wait_scatter(nxt)
                else:
                    wait_scatter(nxt)
                if j == _BLKC - 1:
                    @pl.when(b < _NBLK - 1)
                    def _():
                        load_blk(base_e + (b + 1) * _IB)
                        compute_scat(0, scat[nxt])
                        start_gather(0, rows[nxt])
                else:
                    compute_scat(j + 1, scat[nxt])
                    start_gather(j + 1, rows[nxt])
            return carry

        lax.fori_loop(0, _NBLK, block_body, 0)
        wait_scatter(1)
        plsc.subcore_barrier()

        pltpu.sync_copy(acc.at[pl.ds(s * _ZROWS, _ZROWS)],
                        agg_hbm.at[pl.ds(r * _ACC_ROWS + s * _ZROWS, _ZROWS)])
        plsc.subcore_barrier()


def _sc_aggregate(x, src, cidx, zeros_blk):
    mesh = plsc.VectorSubcoreMesh(core_axis_name="c", subcore_axis_name="s")
    kern = functools.partial(
        pl.kernel,
        mesh=mesh,
        out_type=jax.ShapeDtypeStruct((_R * _ACC_ROWS, _D), jnp.float32),
        scratch_types=[
            pltpu.VMEM_SHARED((_ACC_ROWS, _D), jnp.float32),
            pltpu.VMEM((_IB,), jnp.int32),
            pltpu.VMEM((_IB,), jnp.int32),
            pltpu.VMEM((_CH,), jnp.int32),
            pltpu.VMEM((_CH,), jnp.int32),
            pltpu.VMEM((_CH, _D), jnp.float32),
            pltpu.VMEM((_CH, _D), jnp.float32),
            pltpu.SemaphoreType.DMA,
            pltpu.SemaphoreType.DMA,
        ],
    )(_sc_body)
    return kern(x, src, cidx, zeros_blk)


def _dot(a, b):
    return lax.dot_general(a, b, (((1,), (0,)), ((), ())),
                           precision=lax.Precision.HIGHEST,
                           preferred_element_type=jnp.float32)


def _tc_body(x_ref, agg_ref, wsl_ref, bsl_ref, w1_ref, b1_ref, g_ref, be_ref,
             w2_ref, b2_ref, out_ref):
    r = pl.program_id(0)
    x = x_ref[...]
    h = x + agg_ref[0]
    h = _dot(h, w1_ref[0]) + b1_ref[0]
    mean = jnp.mean(h, axis=0, keepdims=True)
    d = h - mean
    var = jnp.mean(d * d, axis=0, keepdims=True)
    hn = d * lax.rsqrt(var + _BN_EPS) * g_ref[0] + be_ref[0]
    hn = jnp.maximum(hn, 0.0)
    h2 = _dot(hn, w2_ref[0]) + b2_ref[0]

    @pl.when(r == 0)
    def _():
        out_ref[...] = _dot(x, wsl_ref[...]) + bsl_ref[...] + h2

    @pl.when(r != 0)
    def _():
        out_ref[...] = out_ref[...] + h2


def _tc_mlp(x, agg4, W_sl, b_sl, W1, b1, gamma, beta, W2, b2):
    full2 = pl.BlockSpec((_N, _D), lambda r: (0, 0))
    per_rel_vec = pl.BlockSpec((1, 1, _D), lambda r: (r, 0, 0))
    return pl.pallas_call(
        _tc_body,
        grid=(_R,),
        in_specs=[
            full2,
            pl.BlockSpec((1, _N, _D), lambda r: (r, 0, 0)),
            pl.BlockSpec((_D, _D), lambda r: (0, 0)),
            pl.BlockSpec((1, _D), lambda r: (0, 0)),
            pl.BlockSpec((1, _D, _D), lambda r: (r, 0, 0)),
            per_rel_vec,
            per_rel_vec,
            per_rel_vec,
            pl.BlockSpec((1, _D, _D), lambda r: (r, 0, 0)),
            per_rel_vec,
        ],
        out_specs=full2,
        out_shape=jax.ShapeDtypeStruct((_N, _D), jnp.float32),
        compiler_params=pltpu.CompilerParams(vmem_limit_bytes=100 * 1024 * 1024),
    )(x, agg4, W_sl, b_sl.reshape(1, _D),
      W1, b1.reshape(_R, 1, _D), gamma.reshape(_R, 1, _D),
      beta.reshape(_R, 1, _D), W2, b2.reshape(_R, 1, _D))


def kernel(x, edge_index, edge_type, W_sl, b_sl, W1, b1, gamma, beta, W2, b2):
    src = edge_index[0]
    cidx = edge_type * jnp.int32(_N) + edge_index[1]
    zeros_blk = jnp.zeros((_ZROWS, _D), jnp.float32)
    agg = _sc_aggregate(x, src, cidx, zeros_blk)
    agg4 = agg.reshape(_R, _ACC_ROWS, _D)
    return _tc_mlp(x, agg4, W_sl, b_sl, W1, b1, gamma, beta, W2, b2)
